# baseline (speedup 1.0000x reference)
import jax
import jax.numpy as jnp
from jax.experimental import pallas as pl
from jax.experimental.pallas import tpu as pltpu


def kernel(tokens, W_pos):
    batch, seq_len = tokens.shape
    d_model = W_pos.shape[1]
    s_blk = 256
    nblk = seq_len // s_blk

    def _dma_kernel(w_hbm, o_hbm, buf, in_sems, out_sems):
        for i in range(nblk):
            pltpu.make_async_copy(
                w_hbm.at[pl.ds(i * s_blk, s_blk)], buf.at[i], in_sems.at[i]
            ).start()
        for i in range(nblk):
            pltpu.make_async_copy(
                w_hbm.at[pl.ds(i * s_blk, s_blk)], buf.at[i], in_sems.at[i]
            ).wait()
            for b in range(batch):
                pltpu.make_async_copy(
                    buf.at[i], o_hbm.at[b, pl.ds(i * s_blk, s_blk)],
                    out_sems.at[i, b],
                ).start()
        for i in range(nblk):
            for b in range(batch):
                pltpu.make_async_copy(
                    buf.at[i], o_hbm.at[b, pl.ds(i * s_blk, s_blk)],
                    out_sems.at[i, b],
                ).wait()

    out = pl.pallas_call(
        _dma_kernel,
        in_specs=[pl.BlockSpec(memory_space=pl.ANY)],
        out_specs=pl.BlockSpec(memory_space=pl.ANY),
        out_shape=jax.ShapeDtypeStruct((batch, seq_len, d_model), W_pos.dtype),
        scratch_shapes=[
            pltpu.VMEM((nblk, s_blk, d_model), W_pos.dtype),
            pltpu.SemaphoreType.DMA((nblk,)),
            pltpu.SemaphoreType.DMA((nblk, batch)),
        ],
    )(W_pos)
    return out


# speedup vs baseline: 1.0364x; 1.0364x over previous
import jax
import jax.numpy as jnp
from jax.experimental import pallas as pl
from jax.experimental.pallas import tpu as pltpu


def kernel(tokens, W_pos):
    batch, seq_len = tokens.shape
    d_model = W_pos.shape[1]
    s_blk = 1024
    nblk = seq_len // s_blk

    def _dma_kernel(w_hbm, o_hbm, buf, in_sems, out_sems):
        for i in range(nblk):
            pltpu.make_async_copy(
                w_hbm.at[pl.ds(i * s_blk, s_blk)], buf.at[i], in_sems.at[i]
            ).start()
        for i in range(nblk):
            pltpu.make_async_copy(
                w_hbm.at[pl.ds(i * s_blk, s_blk)], buf.at[i], in_sems.at[i]
            ).wait()
            for b in range(batch):
                pltpu.make_async_copy(
                    buf.at[i], o_hbm.at[b, pl.ds(i * s_blk, s_blk)],
                    out_sems.at[i, b],
                ).start()
        for i in range(nblk):
            for b in range(batch):
                pltpu.make_async_copy(
                    buf.at[i], o_hbm.at[b, pl.ds(i * s_blk, s_blk)],
                    out_sems.at[i, b],
                ).wait()

    out = pl.pallas_call(
        _dma_kernel,
        in_specs=[pl.BlockSpec(memory_space=pl.ANY)],
        out_specs=pl.BlockSpec(memory_space=pl.ANY),
        out_shape=jax.ShapeDtypeStruct((batch, seq_len, d_model), W_pos.dtype),
        scratch_shapes=[
            pltpu.VMEM((nblk, s_blk, d_model), W_pos.dtype),
            pltpu.SemaphoreType.DMA((nblk,)),
            pltpu.SemaphoreType.DMA((nblk, batch)),
        ],
    )(W_pos)
    return out
